# baseline (speedup 1.0000x reference)
import functools

import jax
import jax.numpy as jnp
from jax import lax
from jax.experimental import pallas as pl
from jax.experimental.pallas import tpu as pltpu
from jax.experimental.pallas import tpu_sc as plsc

_MARGIN = 0.2
_MOMENTUM = 0.9
_NC = 16
_D = 512
_BLK = 2048

_SC_CORES = 2
_SC_SUBCORES = 16
_SC_W = _SC_CORES * _SC_SUBCORES
_CHUNK = 64


def _tc_body(fl0, fl1, noff, f_ref, rc_ref, fc_ref,
             loss_out, nfc_out, fsum, fcnt, lacc):
    i = pl.program_id(0)
    nsteps = pl.num_programs(0)

    @pl.when(i == 0)
    def _init():
        fsum[...] = jnp.zeros_like(fsum)
        fcnt[...] = jnp.zeros_like(fcnt)
        lacc[...] = jnp.zeros_like(lacc)

    f = f_ref[...]
    fcid = fl0[0] * 4 + fl1[0]
    ncid = lax.rem(fcid + 1 + noff[0], _NC)

    iota_c = lax.broadcasted_iota(jnp.int32, (_NC, _BLK), 0)
    oh_f = (iota_c == fcid).astype(jnp.float32)
    oh_n = (iota_c == ncid).astype(jnp.float32)

    dn = (((1,), (0,)), ((), ()))
    fsum[...] += lax.dot_general(oh_f, f, dn, preferred_element_type=jnp.float32)
    fcnt[...] += jnp.sum(oh_f, axis=1, keepdims=True)

    cp = rc_ref[...] - 1e-6
    gt = lax.dot_general(cp, f, (((1,), (1,)), ((), ())),
                         preferred_element_type=jnp.float32)
    cn2 = jnp.sum(cp * cp, axis=1, keepdims=True)
    term = cn2 - 2.0 * gt
    posd = jnp.sum(oh_f * term, axis=0, keepdims=True)
    negd = jnp.sum(oh_n * term, axis=0, keepdims=True)
    rown = jnp.reshape(jnp.sum(f * f, axis=1), (1, _BLK))
    dpos = jnp.sqrt(jnp.maximum(rown + posd, 0.0))
    dneg = jnp.sqrt(jnp.maximum(rown + negd, 0.0))
    lacc[...] += jnp.sum(jnp.maximum(dpos - dneg + _MARGIN, 0.0), keepdims=True)

    @pl.when(i == nsteps - 1)
    def _fin():
        loss_out[...] = lacc[...] / (nsteps * _BLK)
        fmean = fsum[...] / jnp.maximum(fcnt[...], 1.0)
        fup = _MOMENTUM * fc_ref[...] + (1.0 - _MOMENTUM) * fmean
        nfc_out[...] = jnp.where(fcnt[...] > 0.0, fup, fc_ref[...])


def _run_tc(f, fl0, fl1, noff, rc, fc):
    grid = f.shape[0] // _BLK
    row3 = pl.BlockSpec((1, 1, _BLK), lambda i: (i, 0, 0))
    rows = pl.BlockSpec((_BLK, _D), lambda i: (i, 0))
    full = pl.BlockSpec((_NC, _D), lambda i: (0, 0))
    return pl.pallas_call(
        _tc_body,
        grid=(grid,),
        in_specs=[row3, row3, row3, rows, full, full],
        out_specs=[pl.BlockSpec((1, 1), lambda i: (0, 0)), full],
        out_shape=[
            jax.ShapeDtypeStruct((1, 1), jnp.float32),
            jax.ShapeDtypeStruct((_NC, _D), jnp.float32),
        ],
        scratch_shapes=[
            pltpu.VMEM((_NC, _D), jnp.float32),
            pltpu.VMEM((_NC, 1), jnp.float32),
            pltpu.VMEM((1, 1), jnp.float32),
        ],
    )(fl0, fl1, noff, f, rc, fc)


def _sc_body(r_hbm, l0_hbm, l1_hbm, psum_hbm,
             rows_v, l0_v, l1_v, cid_v, acc_v):
    c = lax.axis_index("c")
    s = lax.axis_index("s")
    wid = s * _SC_CORES + c
    n_rows = r_hbm.shape[0]
    per_w = n_rows // _SC_W
    base = wid * per_w

    z16 = jnp.zeros((16,), jnp.float32)
    for r in range(_NC):
        for k in range(_D // 16):
            acc_v[r, pl.ds(k * 16, 16)] = z16

    for t in range(per_w // _CHUNK):
        off = base + t * _CHUNK
        pltpu.sync_copy(r_hbm.at[pl.ds(off, _CHUNK)], rows_v)
        pltpu.sync_copy(l0_hbm.at[pl.ds(off, _CHUNK)], l0_v)
        pltpu.sync_copy(l1_hbm.at[pl.ds(off, _CHUNK)], l1_v)
        for k in range(_CHUNK // 16):
            sl = pl.ds(k * 16, 16)
            cid_v[sl] = l0_v[sl] * 4 + l1_v[sl]

        def _row(r2, _):
            cidr = cid_v[pl.ds(r2, 16)][0]
            for j in range(_D // 16):
                x = rows_v[r2, pl.ds(j * 16, 16)]
                plsc.addupdate(acc_v.at[cidr, pl.ds(j * 16, 16)], x)
            return 0

        lax.fori_loop(0, _CHUNK, _row, 0, unroll=4)

    pltpu.sync_copy(acc_v, psum_hbm.at[wid])


def _run_sc(r, l0, l1):
    mesh = plsc.VectorSubcoreMesh(core_axis_name="c", subcore_axis_name="s")
    fn = pl.kernel(
        _sc_body,
        out_type=jax.ShapeDtypeStruct((_SC_W, _NC, _D), jnp.float32),
        mesh=mesh,
        scratch_types=[
            pltpu.VMEM((_CHUNK, _D), jnp.float32),
            pltpu.VMEM((_CHUNK,), jnp.int32),
            pltpu.VMEM((_CHUNK,), jnp.int32),
            pltpu.VMEM((_CHUNK + 16,), jnp.int32),
            pltpu.VMEM((_NC, _D), jnp.float32),
        ],
    )
    return fn(r, l0, l1)


def _fin_body(rl0, rl1, psum_ref, rc_ref, out_ref):
    n = rl0.shape[-1]
    rcid = rl0[0] * 4 + rl1[0]
    iota_c = lax.broadcasted_iota(jnp.int32, (_NC, n), 0)
    oh = (iota_c == rcid).astype(jnp.float32)
    cnts = jnp.sum(oh, axis=1, keepdims=True)
    sums = jnp.sum(psum_ref[...], axis=0)
    mean = sums / jnp.maximum(cnts, 1.0)
    upd = _MOMENTUM * rc_ref[...] + (1.0 - _MOMENTUM) * mean
    out_ref[...] = jnp.where(cnts > 0.0, upd, rc_ref[...])


def _run_fin(rl0, rl1, psum, rc):
    return pl.pallas_call(
        _fin_body,
        out_shape=jax.ShapeDtypeStruct((_NC, _D), jnp.float32),
    )(rl0, rl1, psum, rc)


@jax.jit
def _run(r, f, rl0, rl1, fl0, fl1, noff, rc, fc):
    psum = _run_sc(r, rl0, rl1)
    loss, nfc = _run_tc(f, fl0, fl1, noff, rc, fc)
    nrc = _run_fin(rl0.reshape(1, 1, -1), rl1.reshape(1, 1, -1), psum, rc)
    return loss, nrc, nfc


def kernel(real_double_features, fake_double_features, real_double_labels,
           fake_double_labels, real_centroids, fake_centroids, neg_offset):
    n = fake_double_features.shape[0]
    g = n // _BLK
    shp = (g, 1, _BLK)
    fl0 = fake_double_labels[:, 0].reshape(shp)
    fl1 = fake_double_labels[:, 1].reshape(shp)
    noff = neg_offset.reshape(shp)
    rl0 = real_double_labels[:, 0]
    rl1 = real_double_labels[:, 1]
    loss, nrc, nfc = _run(real_double_features, fake_double_features,
                          rl0, rl1, fl0, fl1, noff,
                          real_centroids, fake_centroids)
    return loss.reshape(()), nrc, nfc


# speedup vs baseline: 1.5328x; 1.5328x over previous
import functools

import jax
import jax.numpy as jnp
from jax import lax
from jax.experimental import pallas as pl
from jax.experimental.pallas import tpu as pltpu
from jax.experimental.pallas import tpu_sc as plsc

_MARGIN = 0.2
_MOMENTUM = 0.9
_NC = 16
_D = 512
_BLK = 2048

_SC_CORES = 2
_SC_SUBCORES = 16
_SC_W = _SC_CORES * _SC_SUBCORES
_CHUNK = 64


def _tc_body(fl0, fl1, noff, f_ref, rc_ref, fc_ref,
             loss_out, nfc_out, fsum, fcnt, lacc):
    i = pl.program_id(0)
    nsteps = pl.num_programs(0)

    @pl.when(i == 0)
    def _init():
        fsum[...] = jnp.zeros_like(fsum)
        fcnt[...] = jnp.zeros_like(fcnt)
        lacc[...] = jnp.zeros_like(lacc)

    f = f_ref[...]
    fcid = fl0[0] * 4 + fl1[0]
    ncid = lax.rem(fcid + 1 + noff[0], _NC)

    iota_c = lax.broadcasted_iota(jnp.int32, (_NC, _BLK), 0)
    oh_f = (iota_c == fcid).astype(jnp.float32)
    oh_n = (iota_c == ncid).astype(jnp.float32)

    dn = (((1,), (0,)), ((), ()))
    fsum[...] += lax.dot_general(oh_f, f, dn, preferred_element_type=jnp.float32)
    fcnt[...] += jnp.sum(oh_f, axis=1, keepdims=True)

    cp = rc_ref[...] - 1e-6
    gt = lax.dot_general(cp, f, (((1,), (1,)), ((), ())),
                         preferred_element_type=jnp.float32)
    cn2 = jnp.sum(cp * cp, axis=1, keepdims=True)
    term = cn2 - 2.0 * gt
    posd = jnp.sum(oh_f * term, axis=0, keepdims=True)
    negd = jnp.sum(oh_n * term, axis=0, keepdims=True)
    rown = jnp.reshape(jnp.sum(f * f, axis=1), (1, _BLK))
    dpos = jnp.sqrt(jnp.maximum(rown + posd, 0.0))
    dneg = jnp.sqrt(jnp.maximum(rown + negd, 0.0))
    lacc[...] += jnp.sum(jnp.maximum(dpos - dneg + _MARGIN, 0.0), keepdims=True)

    @pl.when(i == nsteps - 1)
    def _fin():
        loss_out[...] = lacc[...] / (nsteps * _BLK)
        fmean = fsum[...] / jnp.maximum(fcnt[...], 1.0)
        fup = _MOMENTUM * fc_ref[...] + (1.0 - _MOMENTUM) * fmean
        nfc_out[...] = jnp.where(fcnt[...] > 0.0, fup, fc_ref[...])


def _run_tc(f, fl0, fl1, noff, rc, fc):
    grid = f.shape[0] // _BLK
    row3 = pl.BlockSpec((1, 1, _BLK), lambda i: (i, 0, 0))
    rows = pl.BlockSpec((_BLK, _D), lambda i: (i, 0))
    full = pl.BlockSpec((_NC, _D), lambda i: (0, 0))
    return pl.pallas_call(
        _tc_body,
        grid=(grid,),
        in_specs=[row3, row3, row3, rows, full, full],
        out_specs=[pl.BlockSpec((1, 1), lambda i: (0, 0)), full],
        out_shape=[
            jax.ShapeDtypeStruct((1, 1), jnp.float32),
            jax.ShapeDtypeStruct((_NC, _D), jnp.float32),
        ],
        scratch_shapes=[
            pltpu.VMEM((_NC, _D), jnp.float32),
            pltpu.VMEM((_NC, 1), jnp.float32),
            pltpu.VMEM((1, 1), jnp.float32),
        ],
    )(fl0, fl1, noff, f, rc, fc)


def _sc_body(r_hbm, l0_hbm, l1_hbm, psum_hbm,
             rows_v, l0_v, l1_v, cid_v, acc_v):
    c = lax.axis_index("c")
    s = lax.axis_index("s")
    wid = s * _SC_CORES + c
    n_rows = r_hbm.shape[0]
    per_w = n_rows // _SC_W
    base = wid * per_w

    z16 = jnp.zeros((16,), jnp.float32)
    for r in range(_NC):
        for k in range(_D // 16):
            acc_v[r, pl.ds(k * 16, 16)] = z16

    for t in range(per_w // _CHUNK):
        off = base + t * _CHUNK
        pltpu.sync_copy(r_hbm.at[pl.ds(off, _CHUNK)], rows_v)
        pltpu.sync_copy(l0_hbm.at[pl.ds(off, _CHUNK)], l0_v)
        pltpu.sync_copy(l1_hbm.at[pl.ds(off, _CHUNK)], l1_v)
        for k in range(_CHUNK // 16):
            sl = pl.ds(k * 16, 16)
            cid_v[sl] = l0_v[sl] * 4 + l1_v[sl]

        @plsc.parallel_loop(0, _CHUNK, unroll=2)
        def _row(r2):
            cidr = cid_v[pl.ds(r2, 16)][0]
            for j in range(_D // 16):
                x = rows_v[r2, pl.ds(j * 16, 16)]
                plsc.addupdate(acc_v.at[cidr, pl.ds(j * 16, 16)], x)

    pltpu.sync_copy(acc_v, psum_hbm.at[wid])


def _run_sc(r, l0, l1):
    mesh = plsc.VectorSubcoreMesh(core_axis_name="c", subcore_axis_name="s")
    fn = pl.kernel(
        _sc_body,
        out_type=jax.ShapeDtypeStruct((_SC_W, _NC, _D), jnp.float32),
        mesh=mesh,
        scratch_types=[
            pltpu.VMEM((_CHUNK, _D), jnp.float32),
            pltpu.VMEM((_CHUNK,), jnp.int32),
            pltpu.VMEM((_CHUNK,), jnp.int32),
            pltpu.VMEM((_CHUNK + 16,), jnp.int32),
            pltpu.VMEM((_NC, _D), jnp.float32),
        ],
    )
    return fn(r, l0, l1)


def _fin_body(rl0, rl1, psum_ref, rc_ref, out_ref):
    n = rl0.shape[-1]
    rcid = rl0[0] * 4 + rl1[0]
    iota_c = lax.broadcasted_iota(jnp.int32, (_NC, n), 0)
    oh = (iota_c == rcid).astype(jnp.float32)
    cnts = jnp.sum(oh, axis=1, keepdims=True)
    sums = jnp.sum(psum_ref[...], axis=0)
    mean = sums / jnp.maximum(cnts, 1.0)
    upd = _MOMENTUM * rc_ref[...] + (1.0 - _MOMENTUM) * mean
    out_ref[...] = jnp.where(cnts > 0.0, upd, rc_ref[...])


def _run_fin(rl0, rl1, psum, rc):
    return pl.pallas_call(
        _fin_body,
        out_shape=jax.ShapeDtypeStruct((_NC, _D), jnp.float32),
    )(rl0, rl1, psum, rc)


@jax.jit
def _run(r, f, rl0, rl1, fl0, fl1, noff, rc, fc):
    psum = _run_sc(r, rl0, rl1)
    loss, nfc = _run_tc(f, fl0, fl1, noff, rc, fc)
    nrc = _run_fin(rl0.reshape(1, 1, -1), rl1.reshape(1, 1, -1), psum, rc)
    return loss, nrc, nfc


def kernel(real_double_features, fake_double_features, real_double_labels,
           fake_double_labels, real_centroids, fake_centroids, neg_offset):
    n = fake_double_features.shape[0]
    g = n // _BLK
    shp = (g, 1, _BLK)
    fl0 = fake_double_labels[:, 0].reshape(shp)
    fl1 = fake_double_labels[:, 1].reshape(shp)
    noff = neg_offset.reshape(shp)
    rl0 = real_double_labels[:, 0]
    rl1 = real_double_labels[:, 1]
    loss, nrc, nfc = _run(real_double_features, fake_double_features,
                          rl0, rl1, fl0, fl1, noff,
                          real_centroids, fake_centroids)
    return loss.reshape(()), nrc, nfc


# speedup vs baseline: 2.0016x; 1.3058x over previous
import functools

import jax
import jax.numpy as jnp
from jax import lax
from jax.experimental import pallas as pl
from jax.experimental.pallas import tpu as pltpu
from jax.experimental.pallas import tpu_sc as plsc

_MARGIN = 0.2
_MOMENTUM = 0.9
_NC = 16
_D = 512
_BLK = 2048

_SC_CORES = 2
_SC_SUBCORES = 16
_SC_W = _SC_CORES * _SC_SUBCORES
_CHUNK = 64
_N_PER_W = 16384 // _SC_W


def _tc_body(fl0, fl1, noff, f_ref, rc_ref, fc_ref,
             loss_out, nfc_out, fsum, fcnt, lacc):
    i = pl.program_id(0)
    nsteps = pl.num_programs(0)

    @pl.when(i == 0)
    def _init():
        fsum[...] = jnp.zeros_like(fsum)
        fcnt[...] = jnp.zeros_like(fcnt)
        lacc[...] = jnp.zeros_like(lacc)

    f = f_ref[...]
    fcid = fl0[0] * 4 + fl1[0]
    ncid = lax.rem(fcid + 1 + noff[0], _NC)

    iota_c = lax.broadcasted_iota(jnp.int32, (_NC, _BLK), 0)
    oh_f = (iota_c == fcid).astype(jnp.float32)
    oh_n = (iota_c == ncid).astype(jnp.float32)

    dn = (((1,), (0,)), ((), ()))
    fsum[...] += lax.dot_general(oh_f, f, dn, preferred_element_type=jnp.float32)
    fcnt[...] += jnp.sum(oh_f, axis=1, keepdims=True)

    cp = rc_ref[...] - 1e-6
    gt = lax.dot_general(cp, f, (((1,), (1,)), ((), ())),
                         preferred_element_type=jnp.float32)
    cn2 = jnp.sum(cp * cp, axis=1, keepdims=True)
    term = cn2 - 2.0 * gt
    posd = jnp.sum(oh_f * term, axis=0, keepdims=True)
    negd = jnp.sum(oh_n * term, axis=0, keepdims=True)
    rown = jnp.reshape(jnp.sum(f * f, axis=1), (1, _BLK))
    dpos = jnp.sqrt(jnp.maximum(rown + posd, 0.0))
    dneg = jnp.sqrt(jnp.maximum(rown + negd, 0.0))
    lacc[...] += jnp.sum(jnp.maximum(dpos - dneg + _MARGIN, 0.0), keepdims=True)

    @pl.when(i == nsteps - 1)
    def _fin():
        loss_out[...] = lacc[...] / (nsteps * _BLK)
        fmean = fsum[...] / jnp.maximum(fcnt[...], 1.0)
        fup = _MOMENTUM * fc_ref[...] + (1.0 - _MOMENTUM) * fmean
        nfc_out[...] = jnp.where(fcnt[...] > 0.0, fup, fc_ref[...])


def _run_tc(f, fl0, fl1, noff, rc, fc):
    grid = f.shape[0] // _BLK
    row3 = pl.BlockSpec((1, 1, _BLK), lambda i: (i, 0, 0))
    rows = pl.BlockSpec((_BLK, _D), lambda i: (i, 0))
    full = pl.BlockSpec((_NC, _D), lambda i: (0, 0))
    return pl.pallas_call(
        _tc_body,
        grid=(grid,),
        in_specs=[row3, row3, row3, rows, full, full],
        out_specs=[pl.BlockSpec((1, 1), lambda i: (0, 0)), full],
        out_shape=[
            jax.ShapeDtypeStruct((1, 1), jnp.float32),
            jax.ShapeDtypeStruct((_NC, _D), jnp.float32),
        ],
        scratch_shapes=[
            pltpu.VMEM((_NC, _D), jnp.float32),
            pltpu.VMEM((_NC, 1), jnp.float32),
            pltpu.VMEM((1, 1), jnp.float32),
        ],
    )(fl0, fl1, noff, f, rc, fc)


def _sc_body(r_hbm, l0_hbm, l1_hbm, psum_hbm,
             rows_v, l0_v, l1_v, cid_v, acc_v, sem_v):
    c = lax.axis_index("c")
    s = lax.axis_index("s")
    wid = s * _SC_CORES + c
    n_rows = r_hbm.shape[0]
    per_w = n_rows // _SC_W
    base = wid * per_w

    z16 = jnp.zeros((16,), jnp.float32)

    @plsc.parallel_loop(0, _NC, unroll=1)
    def _zero(r):
        for k in range(_D // 16):
            acc_v[r, pl.ds(k * 16, 16)] = z16

    pltpu.sync_copy(l0_hbm.at[pl.ds(base, per_w)], l0_v)
    pltpu.sync_copy(l1_hbm.at[pl.ds(base, per_w)], l1_v)

    @plsc.parallel_loop(0, per_w // 16, unroll=2)
    def _cid(k):
        sl = pl.ds(k * 16, 16)
        cid_v[sl] = l0_v[sl] * 4 + l1_v[sl]

    nbuf = 2

    def _dma(t):
        return pltpu.make_async_copy(
            r_hbm.at[pl.ds(base + t * _CHUNK, _CHUNK)],
            rows_v.at[t % nbuf], sem_v.at[t % nbuf])

    nchunks = per_w // _CHUNK
    _dma(0).start()
    for t in range(nchunks):
        b = t % nbuf
        if t + 1 < nchunks:
            _dma(t + 1).start()
        _dma(t).wait()

        @plsc.parallel_loop(0, _CHUNK, unroll=2)
        def _row(r2):
            cidr = cid_v[pl.ds(t * _CHUNK + r2, 16)][0]
            for j in range(_D // 16):
                x = rows_v[b, r2, pl.ds(j * 16, 16)]
                plsc.addupdate(acc_v.at[cidr, pl.ds(j * 16, 16)], x)

    pltpu.sync_copy(acc_v, psum_hbm.at[wid])


def _run_sc(r, l0, l1):
    mesh = plsc.VectorSubcoreMesh(core_axis_name="c", subcore_axis_name="s")
    fn = pl.kernel(
        _sc_body,
        out_type=jax.ShapeDtypeStruct((_SC_W, _NC, _D), jnp.float32),
        mesh=mesh,
        scratch_types=[
            pltpu.VMEM((2, _CHUNK, _D), jnp.float32),
            pltpu.VMEM((_N_PER_W,), jnp.int32),
            pltpu.VMEM((_N_PER_W,), jnp.int32),
            pltpu.VMEM((_N_PER_W + 16,), jnp.int32),
            pltpu.VMEM((_NC, _D), jnp.float32),
            pltpu.SemaphoreType.DMA((2,)),
        ],
    )
    return fn(r, l0, l1)


def _fin_body(rl0, rl1, psum_ref, rc_ref, out_ref):
    n = rl0.shape[-1]
    rcid = rl0[0] * 4 + rl1[0]
    iota_c = lax.broadcasted_iota(jnp.int32, (_NC, n), 0)
    oh = (iota_c == rcid).astype(jnp.float32)
    cnts = jnp.sum(oh, axis=1, keepdims=True)
    sums = jnp.sum(psum_ref[...], axis=0)
    mean = sums / jnp.maximum(cnts, 1.0)
    upd = _MOMENTUM * rc_ref[...] + (1.0 - _MOMENTUM) * mean
    out_ref[...] = jnp.where(cnts > 0.0, upd, rc_ref[...])


def _run_fin(rl0, rl1, psum, rc):
    return pl.pallas_call(
        _fin_body,
        out_shape=jax.ShapeDtypeStruct((_NC, _D), jnp.float32),
    )(rl0, rl1, psum, rc)


@jax.jit
def _run(r, f, rl0, rl1, fl0, fl1, noff, rc, fc):
    psum = _run_sc(r, rl0, rl1)
    loss, nfc = _run_tc(f, fl0, fl1, noff, rc, fc)
    nrc = _run_fin(rl0.reshape(1, 1, -1), rl1.reshape(1, 1, -1), psum, rc)
    return loss, nrc, nfc


def kernel(real_double_features, fake_double_features, real_double_labels,
           fake_double_labels, real_centroids, fake_centroids, neg_offset):
    n = fake_double_features.shape[0]
    g = n // _BLK
    shp = (g, 1, _BLK)
    fl0 = fake_double_labels[:, 0].reshape(shp)
    fl1 = fake_double_labels[:, 1].reshape(shp)
    noff = neg_offset.reshape(shp)
    rl0 = real_double_labels[:, 0]
    rl1 = real_double_labels[:, 1]
    loss, nrc, nfc = _run(real_double_features, fake_double_features,
                          rl0, rl1, fl0, fl1, noff,
                          real_centroids, fake_centroids)
    return loss.reshape(()), nrc, nfc


# speedup vs baseline: 2.2503x; 1.1243x over previous
import functools

import jax
import jax.numpy as jnp
from jax import lax
from jax.experimental import pallas as pl
from jax.experimental.pallas import tpu as pltpu
from jax.experimental.pallas import tpu_sc as plsc

_MARGIN = 0.2
_MOMENTUM = 0.9
_NC = 16
_D = 512
_BLK = 2048

_SC_CORES = 2
_SC_SUBCORES = 16
_SC_W = _SC_CORES * _SC_SUBCORES
_CHUNK = 64
_N_PER_W = 16384 // _SC_W


def _tc_body(fl0, fl1, noff, f_ref, rc_ref, fc_ref,
             loss_out, nfc_out, fsum, fcnt, lacc):
    i = pl.program_id(0)
    nsteps = pl.num_programs(0)

    @pl.when(i == 0)
    def _init():
        fsum[...] = jnp.zeros_like(fsum)
        fcnt[...] = jnp.zeros_like(fcnt)
        lacc[...] = jnp.zeros_like(lacc)

    f = f_ref[...]
    fcid = fl0[0] * 4 + fl1[0]
    ncid = lax.rem(fcid + 1 + noff[0], _NC)

    iota_c = lax.broadcasted_iota(jnp.int32, (_NC, _BLK), 0)
    oh_f = (iota_c == fcid).astype(jnp.float32)
    oh_n = (iota_c == ncid).astype(jnp.float32)

    dn = (((1,), (0,)), ((), ()))
    fsum[...] += lax.dot_general(oh_f, f, dn, preferred_element_type=jnp.float32)
    fcnt[...] += jnp.sum(oh_f, axis=1, keepdims=True)

    cp = rc_ref[...] - 1e-6
    gt = lax.dot_general(cp, f, (((1,), (1,)), ((), ())),
                         preferred_element_type=jnp.float32)
    cn2 = jnp.sum(cp * cp, axis=1, keepdims=True)
    term = cn2 - 2.0 * gt
    posd = jnp.sum(oh_f * term, axis=0, keepdims=True)
    negd = jnp.sum(oh_n * term, axis=0, keepdims=True)
    rown = jnp.reshape(jnp.sum(f * f, axis=1), (1, _BLK))
    dpos = jnp.sqrt(jnp.maximum(rown + posd, 0.0))
    dneg = jnp.sqrt(jnp.maximum(rown + negd, 0.0))
    lacc[...] += jnp.sum(jnp.maximum(dpos - dneg + _MARGIN, 0.0), keepdims=True)

    @pl.when(i == nsteps - 1)
    def _fin():
        loss_out[...] = lacc[...] / (nsteps * _BLK)
        fmean = fsum[...] / jnp.maximum(fcnt[...], 1.0)
        fup = _MOMENTUM * fc_ref[...] + (1.0 - _MOMENTUM) * fmean
        nfc_out[...] = jnp.where(fcnt[...] > 0.0, fup, fc_ref[...])


def _run_tc(f, fl0, fl1, noff, rc, fc):
    grid = f.shape[0] // _BLK
    row3 = pl.BlockSpec((1, 1, _BLK), lambda i: (i, 0, 0))
    rows = pl.BlockSpec((_BLK, _D), lambda i: (i, 0))
    full = pl.BlockSpec((_NC, _D), lambda i: (0, 0))
    return pl.pallas_call(
        _tc_body,
        grid=(grid,),
        in_specs=[row3, row3, row3, rows, full, full],
        out_specs=[pl.BlockSpec((1, 1), lambda i: (0, 0)), full],
        out_shape=[
            jax.ShapeDtypeStruct((1, 1), jnp.float32),
            jax.ShapeDtypeStruct((_NC, _D), jnp.float32),
        ],
        scratch_shapes=[
            pltpu.VMEM((_NC, _D), jnp.float32),
            pltpu.VMEM((_NC, 1), jnp.float32),
            pltpu.VMEM((1, 1), jnp.float32),
        ],
    )(fl0, fl1, noff, f, rc, fc)


def _sc_body(r_hbm, l0_hbm, l1_hbm, psum_hbm,
             rows_v, l0_v, l1_v, cid_v, acc_v, sem_v):
    c = lax.axis_index("c")
    s = lax.axis_index("s")
    wid = s * _SC_CORES + c
    n_rows = r_hbm.shape[0]
    per_w = n_rows // _SC_W
    base = wid * per_w

    z16 = jnp.zeros((16,), jnp.float32)

    @plsc.parallel_loop(0, _NC, unroll=1)
    def _zero(r):
        for k in range(_D // 16):
            acc_v[r, pl.ds(k * 16, 16)] = z16

    pltpu.sync_copy(l0_hbm.at[pl.ds(base, per_w)], l0_v)
    pltpu.sync_copy(l1_hbm.at[pl.ds(base, per_w)], l1_v)

    @plsc.parallel_loop(0, per_w // 16, unroll=2)
    def _cid(k):
        sl = pl.ds(k * 16, 16)
        cid_v[sl] = l0_v[sl] * 4 + l1_v[sl]

    nchunks = per_w // _CHUNK

    def _dma(t):
        b = lax.rem(t, 2)
        return pltpu.make_async_copy(
            r_hbm.at[pl.ds(base + t * _CHUNK, _CHUNK)],
            rows_v.at[b], sem_v.at[b])

    _dma(0).start()

    def _chunk(t, _):
        b = lax.rem(t, 2)

        @pl.when(t + 1 < nchunks)
        def _pref():
            _dma(t + 1).start()

        _dma(t).wait()

        @plsc.parallel_loop(0, _CHUNK, unroll=4)
        def _row(r2):
            cidr = cid_v[pl.ds(t * _CHUNK + r2, 16)][0]
            for j in range(_D // 16):
                x = rows_v[b, r2, pl.ds(j * 16, 16)]
                plsc.addupdate(acc_v.at[cidr, pl.ds(j * 16, 16)], x)

        return 0

    lax.fori_loop(0, nchunks, _chunk, 0)
    pltpu.sync_copy(acc_v, psum_hbm.at[wid])


def _run_sc(r, l0, l1):
    mesh = plsc.VectorSubcoreMesh(core_axis_name="c", subcore_axis_name="s")
    fn = pl.kernel(
        _sc_body,
        out_type=jax.ShapeDtypeStruct((_SC_W, _NC, _D), jnp.float32),
        mesh=mesh,
        scratch_types=[
            pltpu.VMEM((2, _CHUNK, _D), jnp.float32),
            pltpu.VMEM((_N_PER_W,), jnp.int32),
            pltpu.VMEM((_N_PER_W,), jnp.int32),
            pltpu.VMEM((_N_PER_W + 16,), jnp.int32),
            pltpu.VMEM((_NC, _D), jnp.float32),
            pltpu.SemaphoreType.DMA((2,)),
        ],
    )
    return fn(r, l0, l1)


def _fin_body(rl0, rl1, psum_ref, rc_ref, out_ref):
    n = rl0.shape[-1]
    rcid = rl0[0] * 4 + rl1[0]
    iota_c = lax.broadcasted_iota(jnp.int32, (_NC, n), 0)
    oh = (iota_c == rcid).astype(jnp.float32)
    cnts = jnp.sum(oh, axis=1, keepdims=True)
    sums = jnp.sum(psum_ref[...], axis=0)
    mean = sums / jnp.maximum(cnts, 1.0)
    upd = _MOMENTUM * rc_ref[...] + (1.0 - _MOMENTUM) * mean
    out_ref[...] = jnp.where(cnts > 0.0, upd, rc_ref[...])


def _run_fin(rl0, rl1, psum, rc):
    return pl.pallas_call(
        _fin_body,
        out_shape=jax.ShapeDtypeStruct((_NC, _D), jnp.float32),
    )(rl0, rl1, psum, rc)


@jax.jit
def _run(r, f, rl0, rl1, fl0, fl1, noff, rc, fc):
    psum = _run_sc(r, rl0, rl1)
    loss, nfc = _run_tc(f, fl0, fl1, noff, rc, fc)
    nrc = _run_fin(rl0.reshape(1, 1, -1), rl1.reshape(1, 1, -1), psum, rc)
    return loss, nrc, nfc


def kernel(real_double_features, fake_double_features, real_double_labels,
           fake_double_labels, real_centroids, fake_centroids, neg_offset):
    n = fake_double_features.shape[0]
    g = n // _BLK
    shp = (g, 1, _BLK)
    fl0 = fake_double_labels[:, 0].reshape(shp)
    fl1 = fake_double_labels[:, 1].reshape(shp)
    noff = neg_offset.reshape(shp)
    rl0 = real_double_labels[:, 0]
    rl1 = real_double_labels[:, 1]
    loss, nrc, nfc = _run(real_double_features, fake_double_features,
                          rl0, rl1, fl0, fl1, noff,
                          real_centroids, fake_centroids)
    return loss.reshape(()), nrc, nfc
